# baseline (speedup 1.0000x reference)
import functools

import jax
import jax.numpy as jnp
from jax import lax
from jax.experimental import pallas as pl
from jax.experimental.pallas import tpu as pltpu
from jax.experimental.pallas import tpu_sc as plsc

EMBED = 32
LANES = 16
NC, NS = 2, 16
NW = NC * NS
BBLK = 128


def _sc_embed(idx_flat, table2, pos, seq):
    batch = idx_flat.shape[0] // seq

    mesh = plsc.VectorSubcoreMesh(core_axis_name="c", subcore_axis_name="s")

    @functools.partial(
        pl.kernel,
        out_type=jax.ShapeDtypeStruct((seq, EMBED, batch), jnp.float32),
        mesh=mesh,
        scratch_types=[
            pltpu.VMEM((seq * EMBED,), jnp.float32),
            pltpu.VMEM((BBLK,), jnp.int32),
            pltpu.VMEM((BBLK,), jnp.int32),
            pltpu.VMEM((BBLK, 128), jnp.float32),
            pltpu.VMEM((EMBED, BBLK), jnp.float32),
            pltpu.SemaphoreType.DMA,
        ],
        compiler_params=pltpu.CompilerParams(
            use_tc_tiling_on_sc=False, needs_layout_passes=False
        ),
    )
    def k(idx_hbm, tok_hbm, pos_hbm, out_hbm, pos_v, t_v, r_v, g_v, o_v, sem):
        wid = lax.axis_index("s") * NC + lax.axis_index("c")
        b0 = wid * BBLK
        pltpu.sync_copy(pos_hbm, pos_v)

        def step(s, carry):
            pltpu.sync_copy(idx_hbm.at[pl.ds(s * batch + b0, BBLK)], t_v)
            for l in range(BBLK // LANES):
                tv = t_v[pl.ds(l * LANES, LANES)]
                r_v[pl.ds(l * LANES, LANES)] = lax.shift_right_logical(tv, 2)
            pltpu.async_copy(tok_hbm.at[r_v], g_v, sem).wait()

            sv = lax.broadcast(s * EMBED, (LANES,))
            pos_sp = [
                plsc.load_gather(pos_v, [sv + e]) for e in range(EMBED)
            ]
            for l in range(BBLK // LANES):
                tv = t_v[pl.ds(l * LANES, LANES)]
                colbase = lax.shift_left(tv & 3, 5)
                jv = lax.iota(jnp.int32, LANES) + (l * LANES)
                for e in range(EMBED):
                    val = plsc.load_gather(g_v, [jv, colbase + e])
                    o_v[e, pl.ds(l * LANES, LANES)] = val + pos_sp[e]
            pltpu.sync_copy(o_v, out_hbm.at[s, :, pl.ds(b0, BBLK)])
            return carry

        lax.fori_loop(0, seq, step, 0)

    return k(idx_flat, table2, pos)


def kernel(inputs, token_table, pos_table):
    b, s = inputs.shape
    idx_flat = inputs.astype(jnp.int32).T.reshape(b * s)
    table2 = jnp.reshape(token_table, (-1, 128))
    pos1 = pos_table[:s].reshape(s * EMBED)
    out_t = _sc_embed(idx_flat, table2, pos1, s)
    return out_t.transpose(2, 0, 1)


# speedup vs baseline: 1.4108x; 1.4108x over previous
import functools

import jax
import jax.numpy as jnp
from jax import lax
from jax.experimental import pallas as pl
from jax.experimental.pallas import tpu as pltpu
from jax.experimental.pallas import tpu_sc as plsc

EMBED = 32
LANES = 16
NC, NS = 2, 16
NW = NC * NS
BBLK = 128


def _sc_embed(idx_flat, token_table, pos):
    seq = pos.shape[0]
    batch = idx_flat.shape[0] // seq

    mesh = plsc.VectorSubcoreMesh(core_axis_name="c", subcore_axis_name="s")

    @functools.partial(
        pl.kernel,
        out_type=jax.ShapeDtypeStruct((seq, batch, EMBED), jnp.float32),
        mesh=mesh,
        scratch_types=[
            pltpu.VMEM((seq, EMBED), jnp.float32),
            pltpu.VMEM((BBLK,), jnp.int32),
            pltpu.VMEM((BBLK, EMBED), jnp.float32),
            pltpu.SemaphoreType.DMA,
        ],
        compiler_params=pltpu.CompilerParams(use_tc_tiling_on_sc=False),
    )
    def k(idx_hbm, tok_hbm, pos_hbm, out_hbm, pos_v, t_v, g_v, sem):
        wid = lax.axis_index("s") * NC + lax.axis_index("c")
        b0 = wid * BBLK
        pltpu.sync_copy(pos_hbm, pos_v)

        def step(s, carry):
            pltpu.sync_copy(idx_hbm.at[pl.ds(s * batch + b0, BBLK)], t_v)
            pltpu.async_copy(tok_hbm.at[t_v], g_v, sem).wait()
            pv0 = pos_v[s, pl.ds(0, LANES)]
            pv1 = pos_v[s, pl.ds(LANES, LANES)]
            for j in range(BBLK):
                plsc.addupdate(g_v.at[j, pl.ds(0, LANES)], pv0)
                plsc.addupdate(g_v.at[j, pl.ds(LANES, LANES)], pv1)
            pltpu.sync_copy(g_v, out_hbm.at[s, pl.ds(b0, BBLK)])
            return carry

        lax.fori_loop(0, seq, step, 0)

    return k(idx_flat, token_table, pos)


def kernel(inputs, token_table, pos_table):
    b, s = inputs.shape
    idx_flat = inputs.astype(jnp.int32).T.reshape(b * s)
    out_t = _sc_embed(idx_flat, token_table, pos_table[:s])
    return out_t.transpose(1, 0, 2)


# speedup vs baseline: 1.4528x; 1.0298x over previous
import functools

import jax
import jax.numpy as jnp
from jax import lax
from jax.experimental import pallas as pl
from jax.experimental.pallas import tpu as pltpu
from jax.experimental.pallas import tpu_sc as plsc

EMBED = 32
LANES = 16
NC, NS = 2, 16
NW = NC * NS
BBLK = 128


def _sc_embed(idx_t, table_p, pos):
    seq, batch = idx_t.shape

    mesh = plsc.VectorSubcoreMesh(core_axis_name="c", subcore_axis_name="s")

    @functools.partial(
        pl.kernel,
        out_type=jax.ShapeDtypeStruct((seq, batch, EMBED), jnp.float32),
        mesh=mesh,
        scratch_types=[
            pltpu.VMEM((seq, EMBED), jnp.float32),
            pltpu.VMEM((BBLK,), jnp.int32),
            pltpu.VMEM((BBLK, 128), jnp.float32),
            pltpu.VMEM((BBLK, EMBED), jnp.float32),
            pltpu.SemaphoreType.DMA,
        ],
    )
    def k(idx_hbm, tok_hbm, pos_hbm, out_hbm, pos_v, t_v, g_v, o_v, sem):
        wid = lax.axis_index("s") * NC + lax.axis_index("c")
        b0 = wid * BBLK
        pltpu.sync_copy(pos_hbm, pos_v)

        def step(s, carry):
            pltpu.sync_copy(idx_hbm.at[s, pl.ds(b0, BBLK)], t_v)
            pltpu.async_copy(tok_hbm.at[t_v], g_v, sem).wait()
            pv0 = pos_v[s, pl.ds(0, LANES)]
            pv1 = pos_v[s, pl.ds(LANES, LANES)]
            for j in range(BBLK):
                o_v[j, pl.ds(0, LANES)] = g_v[j, pl.ds(0, LANES)] + pv0
                o_v[j, pl.ds(LANES, LANES)] = g_v[j, pl.ds(LANES, LANES)] + pv1
            pltpu.sync_copy(o_v, out_hbm.at[s, pl.ds(b0, BBLK)])
            return carry

        lax.fori_loop(0, seq, step, 0)

    return k(idx_t, table_p, pos)


def kernel(inputs, token_table, pos_table):
    b, s = inputs.shape
    idx_t = inputs.astype(jnp.int32).T
    table_p = jnp.pad(token_table, ((0, 0), (0, 128 - EMBED)))
    out_t = _sc_embed(idx_t, table_p, pos_table[:s])
    return out_t.transpose(1, 0, 2)


# speedup vs baseline: 1.8117x; 1.2470x over previous
import functools

import jax
import jax.numpy as jnp
from jax import lax
from jax.experimental import pallas as pl
from jax.experimental.pallas import tpu as pltpu
from jax.experimental.pallas import tpu_sc as plsc

EMBED = 32
LANES = 16
NC, NS = 2, 16
NW = NC * NS
BBLK = 128


def _sc_embed(idx_t, table_p, pos):
    seq, batch = idx_t.shape

    mesh = plsc.VectorSubcoreMesh(core_axis_name="c", subcore_axis_name="s")

    @functools.partial(
        pl.kernel,
        out_type=jax.ShapeDtypeStruct((seq, batch, EMBED), jnp.float32),
        mesh=mesh,
        scratch_types=[
            pltpu.VMEM((seq, EMBED), jnp.float32),
            pltpu.VMEM((BBLK,), jnp.int32),
            pltpu.VMEM((BBLK,), jnp.int32),
            pltpu.VMEM((BBLK, 128), jnp.float32),
            pltpu.VMEM((BBLK, 128), jnp.float32),
            pltpu.VMEM((BBLK, EMBED), jnp.float32),
            pltpu.SemaphoreType.DMA,
            pltpu.SemaphoreType.DMA,
        ],
    )
    def k(idx_hbm, tok_hbm, pos_hbm, out_hbm,
          pos_v, t0_v, t1_v, g0_v, g1_v, o_v, sem0, sem1):
        wid = lax.axis_index("s") * NC + lax.axis_index("c")
        b0 = wid * BBLK
        pltpu.sync_copy(pos_hbm, pos_v)

        def emit(s, t_v, g_v):
            pv0 = pos_v[s, pl.ds(0, LANES)]
            pv1 = pos_v[s, pl.ds(LANES, LANES)]
            for j in range(BBLK):
                o_v[j, pl.ds(0, LANES)] = g_v[j, pl.ds(0, LANES)] + pv0
                o_v[j, pl.ds(LANES, LANES)] = g_v[j, pl.ds(LANES, LANES)] + pv1
            pltpu.sync_copy(o_v, out_hbm.at[s, pl.ds(b0, BBLK)])

        pltpu.sync_copy(idx_hbm.at[0, pl.ds(b0, BBLK)], t0_v)
        g0 = pltpu.async_copy(tok_hbm.at[t0_v], g0_v, sem0)

        def pair(i, carry):
            s1 = 2 * i + 1
            pltpu.sync_copy(idx_hbm.at[s1, pl.ds(b0, BBLK)], t1_v)
            g1 = pltpu.async_copy(tok_hbm.at[t1_v], g1_v, sem1)
            pltpu.make_async_copy(tok_hbm.at[t0_v], g0_v, sem0).wait()
            emit(2 * i, t0_v, g0_v)

            @pl.when(i < (seq // 2) - 1)
            def _prefetch_even():
                pltpu.sync_copy(idx_hbm.at[s1 + 1, pl.ds(b0, BBLK)], t0_v)
                pltpu.async_copy(tok_hbm.at[t0_v], g0_v, sem0)

            g1.wait()
            emit(s1, t1_v, g1_v)
            return carry

        lax.fori_loop(0, seq // 2, pair, 0)

    return k(idx_t, table_p, pos)


def kernel(inputs, token_table, pos_table):
    b, s = inputs.shape
    idx_t = inputs.astype(jnp.int32).T
    table_p = jnp.pad(token_table, ((0, 0), (0, 128 - EMBED)))
    out_t = _sc_embed(idx_t, table_p, pos_table[:s])
    return out_t.transpose(1, 0, 2)


# speedup vs baseline: 1.8948x; 1.0458x over previous
import functools

import jax
import jax.numpy as jnp
from jax import lax
from jax.experimental import pallas as pl
from jax.experimental.pallas import tpu as pltpu
from jax.experimental.pallas import tpu_sc as plsc

EMBED = 32
LANES = 16
NC, NS = 2, 16
NW = NC * NS
BBLK = 128


def _sc_embed(idx_t, table_p, pos):
    seq, batch = idx_t.shape

    mesh = plsc.VectorSubcoreMesh(core_axis_name="c", subcore_axis_name="s")

    @functools.partial(
        pl.kernel,
        out_type=jax.ShapeDtypeStruct((seq, batch, EMBED), jnp.float32),
        mesh=mesh,
        scratch_types=[
            pltpu.VMEM((seq, EMBED), jnp.float32),
            pltpu.VMEM((BBLK,), jnp.int32),
            pltpu.VMEM((BBLK,), jnp.int32),
            pltpu.VMEM((BBLK, 128), jnp.float32),
            pltpu.VMEM((BBLK, 128), jnp.float32),
            pltpu.VMEM((BBLK, EMBED), jnp.float32),
            pltpu.VMEM((BBLK, EMBED), jnp.float32),
            pltpu.SemaphoreType.DMA,
            pltpu.SemaphoreType.DMA,
            pltpu.SemaphoreType.DMA,
            pltpu.SemaphoreType.DMA,
        ],
    )
    def k(idx_hbm, tok_hbm, pos_hbm, out_hbm,
          pos_v, t0_v, t1_v, g0_v, g1_v, o0_v, o1_v, sem0, sem1, semo0, semo1):
        wid = lax.axis_index("s") * NC + lax.axis_index("c")
        b0 = wid * BBLK
        pltpu.sync_copy(pos_hbm, pos_v)

        def emit(s, g_v, o_v, semo, drain):
            pv0 = pos_v[s, pl.ds(0, LANES)]
            pv1 = pos_v[s, pl.ds(LANES, LANES)]

            @pl.when(drain)
            def _drain_prev_write():
                pltpu.make_async_copy(
                    o_v, out_hbm.at[s, pl.ds(b0, BBLK)], semo
                ).wait()

            for j in range(BBLK):
                o_v[j, pl.ds(0, LANES)] = g_v[j, pl.ds(0, LANES)] + pv0
                o_v[j, pl.ds(LANES, LANES)] = g_v[j, pl.ds(LANES, LANES)] + pv1
            pltpu.async_copy(o_v, out_hbm.at[s, pl.ds(b0, BBLK)], semo)

        pltpu.sync_copy(idx_hbm.at[0, pl.ds(b0, BBLK)], t0_v)
        g0 = pltpu.async_copy(tok_hbm.at[t0_v], g0_v, sem0)

        def pair(i, carry):
            s1 = 2 * i + 1
            pltpu.sync_copy(idx_hbm.at[s1, pl.ds(b0, BBLK)], t1_v)
            g1 = pltpu.async_copy(tok_hbm.at[t1_v], g1_v, sem1)
            pltpu.make_async_copy(tok_hbm.at[t0_v], g0_v, sem0).wait()
            emit(2 * i, g0_v, o0_v, semo0, i > 0)

            @pl.when(i < (seq // 2) - 1)
            def _prefetch_even():
                pltpu.sync_copy(idx_hbm.at[s1 + 1, pl.ds(b0, BBLK)], t0_v)
                pltpu.async_copy(tok_hbm.at[t0_v], g0_v, sem0)

            g1.wait()
            emit(s1, g1_v, o1_v, semo1, i > 0)
            return carry

        lax.fori_loop(0, seq // 2, pair, 0)
        pltpu.make_async_copy(
            o0_v, out_hbm.at[seq - 2, pl.ds(b0, BBLK)], semo0
        ).wait()
        pltpu.make_async_copy(
            o1_v, out_hbm.at[seq - 1, pl.ds(b0, BBLK)], semo1
        ).wait()

    return k(idx_t, table_p, pos)


def kernel(inputs, token_table, pos_table):
    b, s = inputs.shape
    idx_t = inputs.astype(jnp.int32).T
    table_p = jnp.pad(token_table, ((0, 0), (0, 128 - EMBED)))
    out_t = _sc_embed(idx_t, table_p, pos_table[:s])
    return out_t.transpose(1, 0, 2)
